# baseline (speedup 1.0000x reference)
import functools

import jax
import jax.numpy as jnp
from jax import lax
from jax.experimental import pallas as pl
from jax.experimental.pallas import tpu as pltpu
from jax.experimental.pallas import tpu_sc as plsc

_XDIM = 64
_BSZ = 16384
_NC = 2
_NS = 16
_NW = _NC * _NS
_BPW = _BSZ // _NW
_CHUNK = 128
_NCHUNK = _BPW // _CHUNK

_mesh = plsc.VectorSubcoreMesh(core_axis_name="c", subcore_axis_name="s")


@functools.partial(
    pl.kernel,
    mesh=_mesh,
    compiler_params=pltpu.CompilerParams(use_tc_tiling_on_sc=False),
    out_type=(
        jax.ShapeDtypeStruct((_BSZ, _XDIM), jnp.float32),
        jax.ShapeDtypeStruct((_BSZ,), jnp.int32),
    ),
    scratch_types=[
        pltpu.VMEM((_NCHUNK, _CHUNK), jnp.int32),
        pltpu.VMEM((_BPW, _XDIM), jnp.float32),
        pltpu.VMEM((_BPW,), jnp.int32),
        pltpu.SemaphoreType.DMA,
        pltpu.SemaphoreType.DMA,
    ],
)
def _fetch(idx_hbm, mx_hbm, my_hbm, out_x, out_y, idx_v, rows_v, y_v,
           sem_x, sem_y):
    wid = lax.axis_index("s") * _NC + lax.axis_index("c")
    base = wid * _BPW
    pltpu.sync_copy(idx_hbm.at[pl.ds(wid * _NCHUNK, _NCHUNK)], idx_v)
    copies = []
    for j in range(_NCHUNK):
        copies.append(
            pltpu.async_copy(
                mx_hbm.at[idx_v.at[j]],
                rows_v.at[pl.ds(j * _CHUNK, _CHUNK)],
                sem_x,
            )
        )
        copies.append(
            pltpu.async_copy(
                my_hbm.at[idx_v.at[j]],
                y_v.at[pl.ds(j * _CHUNK, _CHUNK)],
                sem_y,
            )
        )
    for c in copies:
        c.wait()
    pltpu.sync_copy(rows_v, out_x.at[pl.ds(base, _BPW)])
    pltpu.sync_copy(y_v, out_y.at[pl.ds(base, _BPW)])


def kernel(inputs, idx, mems_x, mems_y):
    del inputs
    idx2 = idx.reshape(_NW * _NCHUNK, _CHUNK)
    res_x, res_y = _fetch(idx2, mems_x, mems_y)
    return (res_x, res_y)
